# Initial kernel scaffold; baseline (speedup 1.0000x reference)
import jax
import jax.numpy as jnp
from jax.experimental import pallas as pl


def kernel(x_data, table):
    raise NotImplementedError("write your pallas kernel here")


# speedup vs baseline: 1.3764x; 1.3764x over previous
import jax
import jax.numpy as jnp
from jax import lax
from jax.experimental import pallas as pl
from jax.experimental.pallas import tpu as pltpu
from jax.experimental.pallas import tpu_sc as plsc

DIM = 32
TOTAL = 1_638_400
LANES = 128
ROWS = TOTAL // LANES
NC, NS = 2, 16
NW = NC * NS
ROWS_PER_W = ROWS // NW
K = 8
N_CHUNKS = ROWS_PER_W // K


def _gather_body(table_hbm, idx_hbm, out_hbm, idx_v, rows_v, sem):
    wid = lax.axis_index("s") * NC + lax.axis_index("c")
    base0 = wid * ROWS_PER_W

    def chunk(c, carry):
        base = base0 + c * K
        pltpu.sync_copy(idx_hbm.at[pl.ds(base, K)], idx_v)
        cps = [
            pltpu.async_copy(table_hbm.at[idx_v.at[j]], rows_v.at[j], sem)
            for j in range(K)
        ]
        for cp in cps:
            cp.wait()
        pltpu.sync_copy(rows_v, out_hbm.at[pl.ds(base, K)])
        return carry

    lax.fori_loop(0, N_CHUNKS, chunk, 0)


def kernel(x_data, table):
    idx2d = x_data.astype(jnp.int32).reshape(ROWS, LANES)
    mesh = plsc.VectorSubcoreMesh(core_axis_name="c", subcore_axis_name="s")
    f = pl.kernel(
        _gather_body,
        mesh=mesh,
        out_type=jax.ShapeDtypeStruct((ROWS, LANES, DIM), jnp.float32),
        scratch_types=[
            pltpu.VMEM((K, LANES), jnp.int32),
            pltpu.VMEM((K, LANES, DIM), jnp.float32),
            pltpu.SemaphoreType.DMA,
        ],
        compiler_params=pltpu.CompilerParams(use_tc_tiling_on_sc=False),
    )
    out = f(table, idx2d)
    return out.reshape(TOTAL, DIM)


# speedup vs baseline: 1.4297x; 1.0387x over previous
import jax
import jax.numpy as jnp
from jax import lax
from jax.experimental import pallas as pl
from jax.experimental.pallas import tpu as pltpu
from jax.experimental.pallas import tpu_sc as plsc

DIM = 32
TOTAL = 1_638_400
LANES = 128
ROWS = TOTAL // LANES
NC, NS = 2, 16
NW = NC * NS
ROWS_PER_W = ROWS // NW
K = 8
N_CHUNKS = ROWS_PER_W // K


NBUF = 2


def _gather_body(table_hbm, idx_hbm, out_hbm, idx_v, rows_v, gsems, ssems):
    wid = lax.axis_index("s") * NC + lax.axis_index("c")
    base0 = wid * ROWS_PER_W

    def fire(b, c):
        base = base0 + c * K
        pltpu.sync_copy(idx_hbm.at[pl.ds(base, K)], idx_v.at[b])
        for j in range(K):
            pltpu.async_copy(
                table_hbm.at[idx_v.at[b].at[j]], rows_v.at[b].at[j], gsems.at[b]
            )

    def drain_gathers(b):
        pltpu.make_async_copy(
            out_hbm.at[pl.ds(0, K)], rows_v.at[b], gsems.at[b]
        ).wait()

    for b in range(NBUF):
        fire(b, b)

    def outer(g, carry):
        c0 = g * NBUF
        for b in range(NBUF):
            drain_gathers(b)
            base = base0 + (c0 + b) * K
            pltpu.async_copy(rows_v.at[b], out_hbm.at[pl.ds(base, K)], ssems.at[b])
        for b in range(NBUF):
            pltpu.make_async_copy(
                rows_v.at[b], out_hbm.at[pl.ds(0, K)], ssems.at[b]
            ).wait()

            @pl.when(c0 + b + NBUF < N_CHUNKS)
            def _():
                fire(b, c0 + b + NBUF)

        return carry

    lax.fori_loop(0, N_CHUNKS // NBUF, outer, 0)


def kernel(x_data, table):
    idx2d = x_data.astype(jnp.int32).reshape(ROWS, LANES)
    mesh = plsc.VectorSubcoreMesh(core_axis_name="c", subcore_axis_name="s")
    f = pl.kernel(
        _gather_body,
        mesh=mesh,
        out_type=jax.ShapeDtypeStruct((ROWS, LANES, DIM), jnp.float32),
        scratch_types=[
            pltpu.VMEM((NBUF, K, LANES), jnp.int32),
            pltpu.VMEM((NBUF, K, LANES, DIM), jnp.float32),
            pltpu.SemaphoreType.DMA((NBUF,)),
            pltpu.SemaphoreType.DMA((NBUF,)),
        ],
        compiler_params=pltpu.CompilerParams(use_tc_tiling_on_sc=False),
    )
    out = f(table, idx2d)
    return out.reshape(TOTAL, DIM)
